# baseline (device time: 13537 ns/iter reference)
import jax
import jax.numpy as jnp
from jax import lax
from jax.experimental import pallas as pl
from jax.experimental.pallas import tpu as pltpu


def kernel(x, W, labels):
    T, D = x.shape
    _, V = W.shape
    labels2d = labels.reshape(T, 1)

    def body(x_ref, w_ref, l_ref, out_ref, comm_send, comm_recv,
             send_sem, recv_sem):
        my_x = lax.axis_index("x")
        my_y = lax.axis_index("y")
        my_z = lax.axis_index("z")
        partner = (1 - my_x, my_y, my_z)

        barrier = pltpu.get_barrier_semaphore()
        pl.semaphore_signal(barrier, inc=1, device_id=partner,
                            device_id_type=pl.DeviceIdType.MESH)

        logits = jnp.dot(x_ref[:, :], w_ref[:, :],
                         preferred_element_type=jnp.float32)
        m = jnp.max(logits, axis=1, keepdims=True)
        s = jnp.sum(jnp.exp(logits - m), axis=1, keepdims=True)
        col = lax.broadcasted_iota(jnp.int32, (T, V), 1) + my_x * V
        ll = jnp.sum(jnp.where(col == l_ref[:, :], logits, 0.0),
                     axis=1, keepdims=True)
        comm_send[:, :] = jnp.concatenate(
            [m, s, ll, jnp.zeros((T, 5), jnp.float32)], axis=1)

        pl.semaphore_wait(barrier, 1)

        rdma = pltpu.make_async_remote_copy(
            src_ref=comm_send,
            dst_ref=comm_recv,
            send_sem=send_sem,
            recv_sem=recv_sem,
            device_id=partner,
            device_id_type=pl.DeviceIdType.MESH,
        )
        rdma.start()
        rdma.wait()

        m_r = comm_recv[:, 0:1]
        s_r = comm_recv[:, 1:2]
        ll_r = comm_recv[:, 2:3]
        m_all = jnp.maximum(m, m_r)
        s_all = s * jnp.exp(m - m_all) + s_r * jnp.exp(m_r - m_all)
        out_ref[:, :] = m_all + jnp.log(s_all) - (ll + ll_r)

    out = pl.pallas_call(
        body,
        out_shape=jax.ShapeDtypeStruct((T, 1), jnp.float32),
        in_specs=[
            pl.BlockSpec(memory_space=pltpu.VMEM),
            pl.BlockSpec(memory_space=pltpu.VMEM),
            pl.BlockSpec(memory_space=pltpu.VMEM),
        ],
        out_specs=pl.BlockSpec(memory_space=pltpu.VMEM),
        scratch_shapes=[
            pltpu.VMEM((T, 8), jnp.float32),
            pltpu.VMEM((T, 8), jnp.float32),
            pltpu.SemaphoreType.DMA,
            pltpu.SemaphoreType.DMA,
        ],
        compiler_params=pltpu.CompilerParams(collective_id=0),
    )(x, W, labels2d)
    return out.reshape(T)
